# Initial kernel scaffold; baseline (speedup 1.0000x reference)
import jax
import jax.numpy as jnp
from jax.experimental import pallas as pl


def kernel(x, r, edge_index, Wa, ba, Wd1, bd1, Wd2, bd2):
    raise NotImplementedError("write your pallas kernel here")


# speedup vs baseline: 1.3093x; 1.3093x over previous
import functools

import jax
import jax.numpy as jnp
from jax import lax
from jax.experimental import pallas as pl
from jax.experimental.pallas import tpu as pltpu
from jax.experimental.pallas import tpu_sc as plsc

N_NODES = 10000
N_EDGES = 320000
D = 64
CUTOFF = 5.0

TE = 2000
CH = 128
NCHUNK = N_EDGES // CH
NC, NS = 2, 16
NW = NC * NS
JMAX = (NCHUNK + NW - 1) // NW
ROWS_PER_SUB = N_NODES // NS


def _x1_body(x_ref, wa_ref, ba_ref, o_ref):
    o_ref[...] = (
        jnp.dot(x_ref[...], wa_ref[...], preferred_element_type=jnp.float32)
        + ba_ref[...]
    )


def _wfilt_body(r_ref, wd1_ref, bd1_ref, wd2_ref, bd2_ref, o_ref):
    rr = r_ref[...]
    centers = lax.broadcasted_iota(jnp.int32, (TE, 300), 1).astype(jnp.float32) * 0.1
    diff = rr - centers
    rbf = jnp.exp(-10.0 * diff * diff)
    h = jnp.tanh(
        jnp.dot(rbf, wd1_ref[...], preferred_element_type=jnp.float32)
        + bd1_ref[...]
    )
    h = jnp.tanh(
        jnp.dot(h, wd2_ref[...], preferred_element_type=jnp.float32)
        + bd2_ref[...]
    )
    cut = 0.5 * (jnp.cos(rr * (jnp.pi / CUTOFF)) + 1.0)
    cut = jnp.where(rr < CUTOFF, cut, 0.0)
    o_ref[...] = h * cut


def _out_body(p_ref, wa_ref, ba_ref, o_ref):
    prop = p_ref[0] + p_ref[1]
    x2 = jnp.tanh(
        jnp.dot(prop, wa_ref[...], preferred_element_type=jnp.float32)
        + ba_ref[...]
    )
    o_ref[...] = (
        jnp.dot(x2, wa_ref[...], preferred_element_type=jnp.float32)
        + ba_ref[...]
    )


def _sc_body(x1_hbm, src_hbm, dst_hbm, wf_hbm, zeros_hbm, out_hbm,
             idx_s, idx_d, rows_v, wf_v, acc):
    c = lax.axis_index("c")
    s = lax.axis_index("s")
    wid = s * NC + c

    pltpu.sync_copy(zeros_hbm, acc.at[pl.ds(s * ROWS_PER_SUB, ROWS_PER_SUB)])
    plsc.subcore_barrier()

    def chunk(j, carry):
        g = wid + NW * j

        @pl.when(g < NCHUNK)
        def _():
            base = pl.multiple_of(g * CH, CH)
            pltpu.sync_copy(src_hbm.at[pl.ds(base, CH)], idx_s)
            pltpu.sync_copy(dst_hbm.at[pl.ds(base, CH)], idx_d)
            pltpu.sync_copy(wf_hbm.at[pl.ds(base, CH)], wf_v)
            pltpu.sync_copy(x1_hbm.at[idx_s], rows_v)

            def mul(r_i, mc):
                for q in range(D // 16):
                    sl = pl.ds(q * 16, 16)
                    rows_v[r_i, sl] = rows_v[r_i, sl] * wf_v[r_i, sl]
                return mc

            lax.fori_loop(0, CH, mul, 0)
            pltpu.sync_copy(rows_v, acc.at[idx_d], add=True)

        return carry

    lax.fori_loop(0, JMAX, chunk, 0)
    plsc.subcore_barrier()
    pltpu.sync_copy(
        acc.at[pl.ds(s * ROWS_PER_SUB, ROWS_PER_SUB)],
        out_hbm.at[c * NS + s],
    )


def kernel(x, r, edge_index, Wa, ba, Wd1, bd1, Wd2, bd2):
    f32 = jnp.float32
    ba2 = ba.reshape(1, D)
    bd1_2 = bd1.reshape(1, D)
    bd2_2 = bd2.reshape(1, D)

    x1 = pl.pallas_call(
        _x1_body,
        out_shape=jax.ShapeDtypeStruct((N_NODES, D), f32),
    )(x, Wa, ba2)

    r2 = r.reshape(N_EDGES, 1)
    wfilt = pl.pallas_call(
        _wfilt_body,
        grid=(N_EDGES // TE,),
        in_specs=[
            pl.BlockSpec((TE, 1), lambda i: (i, 0)),
            pl.BlockSpec((300, D), lambda i: (0, 0)),
            pl.BlockSpec((1, D), lambda i: (0, 0)),
            pl.BlockSpec((D, D), lambda i: (0, 0)),
            pl.BlockSpec((1, D), lambda i: (0, 0)),
        ],
        out_specs=pl.BlockSpec((TE, D), lambda i: (i, 0)),
        out_shape=jax.ShapeDtypeStruct((N_EDGES, D), f32),
    )(r2, Wd1, bd1_2, Wd2, bd2_2)

    src = edge_index[0]
    dst = edge_index[1]
    zeros = jnp.zeros((ROWS_PER_SUB, D), f32)

    mesh = plsc.VectorSubcoreMesh(
        core_axis_name="c", subcore_axis_name="s",
        num_cores=NC, num_subcores=NS,
    )
    partials = pl.kernel(
        _sc_body,
        out_type=jax.ShapeDtypeStruct((NC * NS, ROWS_PER_SUB, D), f32),
        mesh=mesh,
        scratch_types=[
            pltpu.VMEM((CH,), jnp.int32),
            pltpu.VMEM((CH,), jnp.int32),
            pltpu.VMEM((CH, D), f32),
            pltpu.VMEM((CH, D), f32),
            pltpu.VMEM_SHARED((N_NODES, D), f32),
        ],
        compiler_params=pltpu.CompilerParams(use_tc_tiling_on_sc=False),
    )(x1, src, dst, wfilt, zeros)
    partials = partials.reshape(NC, N_NODES, D)

    out = pl.pallas_call(
        _out_body,
        out_shape=jax.ShapeDtypeStruct((N_NODES, D), f32),
    )(partials, Wa, ba2)
    return out


# speedup vs baseline: 4.6212x; 3.5296x over previous
import jax
import jax.numpy as jnp
from jax import lax
from jax.experimental import pallas as pl
from jax.experimental.pallas import tpu as pltpu
from jax.experimental.pallas import tpu_sc as plsc

N_NODES = 10000
N_EDGES = 320000
D = 64
CUTOFF = 5.0

NEP = 335872
NEPH = NEP // 2
TE = 2048
NBH = NEPH // TE
CH = 128
WROWS = CH // 2
NCEN = 128
NC, NS = 2, 16
NW = NC * NS
JW = NEPH // WROWS // NW
ROWS_PER_SUB = N_NODES // NS


def _x1_body(x_ref, wa_ref, ba_ref, o_ref):
    o_ref[...] = (
        jnp.dot(x_ref[...], wa_ref[...], preferred_element_type=jnp.float32)
        + ba_ref[...]
    )


def _filter_half(r_row, wd1, bd1, wd2, bd2):
    cut_row = 0.5 * (jnp.cos(r_row * (jnp.pi / CUTOFF)) + 1.0)
    cut_row = jnp.where(r_row < CUTOFF, cut_row, 0.0)
    a = jnp.concatenate([r_row, cut_row], axis=0)
    t = jnp.transpose(a)
    rr = t[:, 0:1]
    cut = t[:, 1:2]
    centers = lax.broadcasted_iota(jnp.int32, (TE, NCEN), 1).astype(jnp.float32) * 0.1
    diff = rr - centers
    rbf = jnp.exp(-10.0 * diff * diff)
    h = jnp.tanh(jnp.dot(rbf, wd1, preferred_element_type=jnp.float32) + bd1)
    h = jnp.tanh(jnp.dot(h, wd2, preferred_element_type=jnp.float32) + bd2)
    return h * cut


def _wfilt_body(ra_ref, rb_ref, wd1_ref, bd1_ref, wd2_ref, bd2_ref, o_ref):
    wd1, bd1 = wd1_ref[...], bd1_ref[...]
    wd2, bd2 = wd2_ref[...], bd2_ref[...]
    ha = _filter_half(ra_ref[0], wd1, bd1, wd2, bd2)
    hb = _filter_half(rb_ref[0], wd1, bd1, wd2, bd2)
    o_ref[...] = jnp.concatenate([ha, hb], axis=1)


def _out_body(p_ref, wa_ref, ba_ref, o_ref):
    prop = p_ref[0] + p_ref[1]
    x2 = jnp.tanh(
        jnp.dot(prop, wa_ref[...], preferred_element_type=jnp.float32)
        + ba_ref[...]
    )
    o_ref[...] = (
        jnp.dot(x2, wa_ref[...], preferred_element_type=jnp.float32)
        + ba_ref[...]
    )


def _sc_body(x1_hbm, src_hbm, dst_hbm, wf_hbm, zeros_hbm, out_hbm,
             idx_s0, idx_s1, idx_s2, idx_d0, idx_d1, idx_d2,
             rows0, rows1, rows2, wf0, wf1, wf2,
             x1s, acc,
             seml0, seml1, seml2, semg0, semg1, semg2, sems0, sems1, sems2):
    c = lax.axis_index("c")
    s = lax.axis_index("s")
    wid = s * NC + c
    half = wid // 16
    wsub = wid % 16

    idx_s = (idx_s0, idx_s1, idx_s2)
    idx_d = (idx_d0, idx_d1, idx_d2)
    rows = (rows0, rows1, rows2)
    wf = (wf0, wf1, wf2)
    seml = (seml0, seml1, seml2)
    semg = (semg0, semg1, semg2)
    sems = (sems0, sems1, sems2)

    def lin_copies(k, st):
        cbase = pl.multiple_of((wsub * JW + k) * CH, CH)
        eb = pl.multiple_of(half * NEPH + cbase, CH)
        return (
            (src_hbm.at[pl.ds(eb, CH)], idx_s[st], seml[st]),
            (dst_hbm.at[pl.ds(eb, CH)], idx_d[st], seml[st]),
            (wf_hbm.at[pl.ds(cbase, CH), pl.ds(half * D, D)], wf[st], seml[st]),
        )

    def lin_issue(k, st):
        for a, b, sm in lin_copies(k, st):
            pltpu.async_copy(a, b, sm)

    def lin_wait(k, st):
        for a, b, sm in lin_copies(k, st):
            pltpu.make_async_copy(a, b, sm).wait()

    lin_issue(0, 0)
    lin_issue(1, 1)
    rsl = pl.ds(s * ROWS_PER_SUB, ROWS_PER_SUB)
    pltpu.sync_copy(x1_hbm.at[rsl], x1s.at[rsl])
    pltpu.sync_copy(zeros_hbm, acc.at[rsl])
    plsc.subcore_barrier()

    def process(kk, k, st):
        lin_wait(k, st)
        pltpu.async_copy(x1s.at[idx_s[st]], rows[st], semg[st])

        st2 = (st + 2) % 3

        @pl.when(kk + (1 if st > 0 else 0) > 0)
        def _():
            pltpu.make_async_copy(rows[st2], acc.at[idx_d[st2]], sems[st2]).wait()

        @pl.when(k + 2 < JW)
        def _():
            lin_issue(k + 2, st2)

        pltpu.make_async_copy(x1s.at[idx_s[st]], rows[st], semg[st]).wait()

        @plsc.parallel_loop(0, CH)
        def _(rp):
            for q in range(4):
                sl = pl.ds(q * 16, 16)
                rows[st][rp, sl] = rows[st][rp, sl] * wf[st][rp, sl]

        pltpu.async_copy(rows[st], acc.at[idx_d[st]], sems[st], add=True)

    def triple(kk, carry):
        for st in range(3):
            process(kk, 3 * kk + st, st)
        return carry

    lax.fori_loop(0, JW // 3, triple, 0)
    process(jnp.int32(JW // 3), jnp.int32(JW - 1), (JW - 1) % 3)
    lst = (JW - 1) % 3
    pltpu.make_async_copy(rows[lst], acc.at[idx_d[lst]], sems[lst]).wait()
    plsc.subcore_barrier()
    pltpu.sync_copy(acc.at[rsl], out_hbm.at[c * NS + s])


def kernel(x, r, edge_index, Wa, ba, Wd1, bd1, Wd2, bd2):
    f32 = jnp.float32
    ba2 = ba.reshape(1, D)
    bd1_2 = bd1.reshape(1, D)
    bd2_2 = bd2.reshape(1, D)

    x1 = pl.pallas_call(
        _x1_body,
        out_shape=jax.ShapeDtypeStruct((N_NODES, D), f32),
    )(x, Wa, ba2)

    npad = NEP - N_EDGES
    r3 = jnp.concatenate([r, jnp.full((npad,), 10.0, f32)]).reshape(NEP // TE, 1, TE)
    wfilt2 = pl.pallas_call(
        _wfilt_body,
        grid=(NBH,),
        in_specs=[
            pl.BlockSpec((1, 1, TE), lambda i: (i, 0, 0)),
            pl.BlockSpec((1, 1, TE), lambda i: (i + NBH, 0, 0)),
            pl.BlockSpec((NCEN, D), lambda i: (0, 0)),
            pl.BlockSpec((1, D), lambda i: (0, 0)),
            pl.BlockSpec((D, D), lambda i: (0, 0)),
            pl.BlockSpec((1, D), lambda i: (0, 0)),
        ],
        out_specs=pl.BlockSpec((TE, 128), lambda i: (i, 0)),
        out_shape=jax.ShapeDtypeStruct((NEPH, 128), f32),
    )(r3, r3, Wd1[:NCEN], bd1_2, Wd2, bd2_2)

    ipad = (jnp.arange(npad, dtype=edge_index.dtype) * 31) % N_NODES
    src = jnp.concatenate([edge_index[0], ipad])
    dst = jnp.concatenate([edge_index[1], ipad])
    zeros = jnp.zeros((ROWS_PER_SUB, D), f32)

    mesh = plsc.VectorSubcoreMesh(
        core_axis_name="c", subcore_axis_name="s",
        num_cores=NC, num_subcores=NS,
    )
    partials = pl.kernel(
        _sc_body,
        out_type=jax.ShapeDtypeStruct((NC * NS, ROWS_PER_SUB, D), f32),
        mesh=mesh,
        scratch_types=[
            pltpu.VMEM((CH,), jnp.int32),
            pltpu.VMEM((CH,), jnp.int32),
            pltpu.VMEM((CH,), jnp.int32),
            pltpu.VMEM((CH,), jnp.int32),
            pltpu.VMEM((CH,), jnp.int32),
            pltpu.VMEM((CH,), jnp.int32),
            pltpu.VMEM((CH, D), f32),
            pltpu.VMEM((CH, D), f32),
            pltpu.VMEM((CH, D), f32),
            pltpu.VMEM((CH, D), f32),
            pltpu.VMEM((CH, D), f32),
            pltpu.VMEM((CH, D), f32),
            pltpu.VMEM_SHARED((N_NODES, D), f32),
            pltpu.VMEM_SHARED((N_NODES, D), f32),
            pltpu.SemaphoreType.DMA,
            pltpu.SemaphoreType.DMA,
            pltpu.SemaphoreType.DMA,
            pltpu.SemaphoreType.DMA,
            pltpu.SemaphoreType.DMA,
            pltpu.SemaphoreType.DMA,
            pltpu.SemaphoreType.DMA,
            pltpu.SemaphoreType.DMA,
            pltpu.SemaphoreType.DMA,
        ],
        compiler_params=pltpu.CompilerParams(use_tc_tiling_on_sc=False),
    )(x1, src, dst, wfilt2, zeros)
    partials = partials.reshape(NC, N_NODES, D)

    out = pl.pallas_call(
        _out_body,
        out_shape=jax.ShapeDtypeStruct((N_NODES, D), f32),
    )(partials, Wa, ba2)
    return out


# speedup vs baseline: 5.2693x; 1.1402x over previous
import jax
import jax.numpy as jnp
from jax import lax
from jax.experimental import pallas as pl
from jax.experimental.pallas import tpu as pltpu
from jax.experimental.pallas import tpu_sc as plsc

N_NODES = 10000
N_EDGES = 320000
D = 64
CUTOFF = 5.0

NEP = 335872
NEPH = NEP // 2
TE = 2048
NBH = NEPH // TE
CH = 128
WROWS = CH // 2
NCEN = 128
NC, NS = 2, 16
NW = NC * NS
JW = NEPH // WROWS // NW
NBA, NBB = 42, 40
JWA, JWB = 42, 40
G0B = NBA * TE // CH
ROWS_PER_SUB = N_NODES // NS


def _x1_body(x_ref, wa_ref, ba_ref, o_ref):
    o_ref[...] = (
        jnp.dot(x_ref[...], wa_ref[...], preferred_element_type=jnp.float32)
        + ba_ref[...]
    )


def _filter_half(r_row, wd1, bd1, wd2, bd2):
    cut_row = 0.5 * (jnp.cos(r_row * (jnp.pi / CUTOFF)) + 1.0)
    cut_row = jnp.where(r_row < CUTOFF, cut_row, 0.0)
    a = jnp.concatenate([r_row, cut_row], axis=0)
    t = jnp.transpose(a)
    rr = t[:, 0:1]
    cut = t[:, 1:2]
    centers = lax.broadcasted_iota(jnp.int32, (TE, NCEN), 1).astype(jnp.float32) * 0.1
    diff = rr - centers
    rbf = jnp.exp(-10.0 * diff * diff)
    h = jnp.tanh(jnp.dot(rbf, wd1, preferred_element_type=jnp.float32) + bd1)
    h = jnp.tanh(jnp.dot(h, wd2, preferred_element_type=jnp.float32) + bd2)
    return h * cut


def _wfilt_body(ra_ref, rb_ref, wd1_ref, bd1_ref, wd2_ref, bd2_ref, o_ref):
    wd1, bd1 = wd1_ref[...], bd1_ref[...]
    wd2, bd2 = wd2_ref[...], bd2_ref[...]
    ha = _filter_half(ra_ref[0], wd1, bd1, wd2, bd2)
    hb = _filter_half(rb_ref[0], wd1, bd1, wd2, bd2)
    o_ref[...] = jnp.concatenate([ha, hb], axis=1)


def _out_body(p_ref, q_ref, wa_ref, ba_ref, o_ref):
    prop = p_ref[0] + p_ref[1] + q_ref[0] + q_ref[1]
    x2 = jnp.tanh(
        jnp.dot(prop, wa_ref[...], preferred_element_type=jnp.float32)
        + ba_ref[...]
    )
    o_ref[...] = (
        jnp.dot(x2, wa_ref[...], preferred_element_type=jnp.float32)
        + ba_ref[...]
    )


def _make_sc_body(g0, jw):
    def _sc_body(x1_hbm, src_hbm, dst_hbm, wf_hbm, zeros_hbm, out_hbm,
                 idx_s0, idx_s1, idx_s2, idx_d0, idx_d1, idx_d2,
                 rows0, rows1, rows2, wf0, wf1, wf2,
                 x1s, acc,
                 seml0, seml1, seml2, semg0, semg1, semg2,
                 sems0, sems1, sems2):
        c = lax.axis_index("c")
        s = lax.axis_index("s")
        wid = s * NC + c
        half = wid // 16
        wsub = wid % 16

        idx_s = (idx_s0, idx_s1, idx_s2)
        idx_d = (idx_d0, idx_d1, idx_d2)
        rows = (rows0, rows1, rows2)
        wf = (wf0, wf1, wf2)
        seml = (seml0, seml1, seml2)
        semg = (semg0, semg1, semg2)
        sems = (sems0, sems1, sems2)

        def lin_copies(k, st):
            cbase = pl.multiple_of((wsub * jw + k) * CH, CH)
            eb = pl.multiple_of(half * NEPH + g0 * CH + cbase, CH)
            return (
                (src_hbm.at[pl.ds(eb, CH)], idx_s[st], seml[st]),
                (dst_hbm.at[pl.ds(eb, CH)], idx_d[st], seml[st]),
                (wf_hbm.at[pl.ds(cbase, CH), pl.ds(half * D, D)], wf[st], seml[st]),
            )

        def lin_issue(k, st):
            for a, b, sm in lin_copies(k, st):
                pltpu.async_copy(a, b, sm)

        def lin_wait(k, st):
            for a, b, sm in lin_copies(k, st):
                pltpu.make_async_copy(a, b, sm).wait()

        lin_issue(0, 0)
        lin_issue(1, 1)
        rsl = pl.ds(s * ROWS_PER_SUB, ROWS_PER_SUB)
        pltpu.sync_copy(x1_hbm.at[rsl], x1s.at[rsl])
        pltpu.sync_copy(zeros_hbm, acc.at[rsl])
        plsc.subcore_barrier()

        def process(kk, k, st):
            lin_wait(k, st)
            pltpu.async_copy(x1s.at[idx_s[st]], rows[st], semg[st])

            st2 = (st + 2) % 3

            @pl.when(kk + (1 if st > 0 else 0) > 0)
            def _():
                pltpu.make_async_copy(rows[st2], acc.at[idx_d[st2]], sems[st2]).wait()

            @pl.when(k + 2 < jw)
            def _():
                lin_issue(k + 2, st2)

            pltpu.make_async_copy(x1s.at[idx_s[st]], rows[st], semg[st]).wait()

            @plsc.parallel_loop(0, CH)
            def _(rp):
                for q in range(4):
                    sl = pl.ds(q * 16, 16)
                    rows[st][rp, sl] = rows[st][rp, sl] * wf[st][rp, sl]

            pltpu.async_copy(rows[st], acc.at[idx_d[st]], sems[st], add=True)

        def triple(kk, carry):
            for st in range(3):
                process(kk, 3 * kk + st, st)
            return carry

        lax.fori_loop(0, jw // 3, triple, 0)
        for t in range(jw % 3):
            k = jw - (jw % 3) + t
            process(jnp.int32(jw // 3), jnp.int32(k), k % 3)
        lst = (jw - 1) % 3
        pltpu.make_async_copy(rows[lst], acc.at[idx_d[lst]], sems[lst]).wait()
        plsc.subcore_barrier()
        pltpu.sync_copy(acc.at[rsl], out_hbm.at[c * NS + s])

    return _sc_body


def kernel(x, r, edge_index, Wa, ba, Wd1, bd1, Wd2, bd2):
    f32 = jnp.float32
    ba2 = ba.reshape(1, D)
    bd1_2 = bd1.reshape(1, D)
    bd2_2 = bd2.reshape(1, D)

    x1 = pl.pallas_call(
        _x1_body,
        out_shape=jax.ShapeDtypeStruct((N_NODES, D), f32),
    )(x, Wa, ba2)

    npad = NEP - N_EDGES
    r3 = jnp.concatenate([r, jnp.full((npad,), 10.0, f32)]).reshape(NEP // TE, 1, TE)

    def wfilt_call(nb, i0):
        return pl.pallas_call(
            _wfilt_body,
            grid=(nb,),
            in_specs=[
                pl.BlockSpec((1, 1, TE), lambda i: (i + i0, 0, 0)),
                pl.BlockSpec((1, 1, TE), lambda i: (i + i0 + NBH, 0, 0)),
                pl.BlockSpec((NCEN, D), lambda i: (0, 0)),
                pl.BlockSpec((1, D), lambda i: (0, 0)),
                pl.BlockSpec((D, D), lambda i: (0, 0)),
                pl.BlockSpec((1, D), lambda i: (0, 0)),
            ],
            out_specs=pl.BlockSpec((TE, 128), lambda i: (i, 0)),
            out_shape=jax.ShapeDtypeStruct((nb * TE, 128), f32),
        )(r3, r3, Wd1[:NCEN], bd1_2, Wd2, bd2_2)

    wf_a = wfilt_call(NBA, 0)
    wf_b = wfilt_call(NBB, NBA)

    ipad = (jnp.arange(npad, dtype=edge_index.dtype) * 31) % N_NODES
    src = jnp.concatenate([edge_index[0], ipad])
    dst = jnp.concatenate([edge_index[1], ipad])
    zeros = jnp.zeros((ROWS_PER_SUB, D), f32)

    mesh = plsc.VectorSubcoreMesh(
        core_axis_name="c", subcore_axis_name="s",
        num_cores=NC, num_subcores=NS,
    )
    scr = (
        [pltpu.VMEM((CH,), jnp.int32)] * 6
        + [pltpu.VMEM((CH, D), f32)] * 6
        + [pltpu.VMEM_SHARED((N_NODES, D), f32)] * 2
        + [pltpu.SemaphoreType.DMA] * 9
    )

    def sc_call(body, wf):
        return pl.kernel(
            body,
            out_type=jax.ShapeDtypeStruct((NC * NS, ROWS_PER_SUB, D), f32),
            mesh=mesh,
            scratch_types=list(scr),
            compiler_params=pltpu.CompilerParams(use_tc_tiling_on_sc=False),
        )(x1, src, dst, wf, zeros)

    p_a = sc_call(_make_sc_body(0, JWA), wf_a)
    p_b = sc_call(_make_sc_body(G0B, JWB), wf_b)
    p_a = p_a.reshape(NC, N_NODES, D)
    p_b = p_b.reshape(NC, N_NODES, D)

    out = pl.pallas_call(
        _out_body,
        out_shape=jax.ShapeDtypeStruct((N_NODES, D), f32),
    )(p_a, p_b, Wa, ba2)
    return out


# speedup vs baseline: 5.3358x; 1.0126x over previous
import jax
import jax.numpy as jnp
from jax import lax
from jax.experimental import pallas as pl
from jax.experimental.pallas import tpu as pltpu
from jax.experimental.pallas import tpu_sc as plsc

N_NODES = 10000
N_EDGES = 320000
D = 64
CUTOFF = 5.0

NEP = 335872
NEPH = NEP // 2
TE = 4096
NBH = NEPH // TE
CH = 128
WROWS = CH // 2
NCEN = 128
NC, NS = 2, 16
NW = NC * NS
JW = NEPH // WROWS // NW
NBA, NBB = 21, 20
JWA, JWB = 42, 40
G0B = NBA * TE // CH
ROWS_PER_SUB = N_NODES // NS


def _x1_body(x_ref, wa_ref, ba_ref, o_ref):
    o_ref[...] = (
        jnp.dot(x_ref[...], wa_ref[...], preferred_element_type=jnp.float32)
        + ba_ref[...]
    )


def _filter_half(r_row, wd1, bd1, wd2, bd2):
    cut_row = 0.5 * (jnp.cos(r_row * (jnp.pi / CUTOFF)) + 1.0)
    cut_row = jnp.where(r_row < CUTOFF, cut_row, 0.0)
    a = jnp.concatenate([r_row, cut_row], axis=0)
    t = jnp.transpose(a)
    rr = t[:, 0:1]
    cut = t[:, 1:2]
    centers = lax.broadcasted_iota(jnp.int32, (TE, NCEN), 1).astype(jnp.float32) * 0.1
    diff = rr - centers
    rbf = jnp.exp(-10.0 * diff * diff)
    h = jnp.tanh(jnp.dot(rbf, wd1, preferred_element_type=jnp.float32) + bd1)
    h = jnp.tanh(jnp.dot(h, wd2, preferred_element_type=jnp.float32) + bd2)
    return h * cut


def _wfilt_body(ra_ref, rb_ref, wd1_ref, bd1_ref, wd2_ref, bd2_ref, o_ref):
    wd1, bd1 = wd1_ref[...], bd1_ref[...]
    wd2, bd2 = wd2_ref[...], bd2_ref[...]
    ha = _filter_half(ra_ref[0], wd1, bd1, wd2, bd2)
    hb = _filter_half(rb_ref[0], wd1, bd1, wd2, bd2)
    o_ref[...] = jnp.concatenate([ha, hb], axis=1)


def _out_body(p_ref, q_ref, wa_ref, ba_ref, o_ref):
    prop = p_ref[0] + p_ref[1] + q_ref[0] + q_ref[1]
    x2 = jnp.tanh(
        jnp.dot(prop, wa_ref[...], preferred_element_type=jnp.float32)
        + ba_ref[...]
    )
    o_ref[...] = (
        jnp.dot(x2, wa_ref[...], preferred_element_type=jnp.float32)
        + ba_ref[...]
    )


def _make_sc_body(g0, jw):
    def _sc_body(x1_hbm, src_hbm, dst_hbm, wf_hbm, zeros_hbm, out_hbm,
                 idx_s0, idx_s1, idx_s2, idx_d0, idx_d1, idx_d2,
                 rows0, rows1, rows2, wf0, wf1, wf2,
                 x1s, acc,
                 seml0, seml1, seml2, semg0, semg1, semg2,
                 sems0, sems1, sems2):
        c = lax.axis_index("c")
        s = lax.axis_index("s")
        wid = s * NC + c
        half = wid // 16
        wsub = wid % 16

        idx_s = (idx_s0, idx_s1, idx_s2)
        idx_d = (idx_d0, idx_d1, idx_d2)
        rows = (rows0, rows1, rows2)
        wf = (wf0, wf1, wf2)
        seml = (seml0, seml1, seml2)
        semg = (semg0, semg1, semg2)
        sems = (sems0, sems1, sems2)

        def lin_copies(k, st):
            cbase = pl.multiple_of((wsub * jw + k) * CH, CH)
            eb = pl.multiple_of(half * NEPH + g0 * CH + cbase, CH)
            return (
                (src_hbm.at[pl.ds(eb, CH)], idx_s[st], seml[st]),
                (dst_hbm.at[pl.ds(eb, CH)], idx_d[st], seml[st]),
                (wf_hbm.at[pl.ds(cbase, CH), pl.ds(half * D, D)], wf[st], seml[st]),
            )

        def lin_issue(k, st):
            for a, b, sm in lin_copies(k, st):
                pltpu.async_copy(a, b, sm)

        def lin_wait(k, st):
            for a, b, sm in lin_copies(k, st):
                pltpu.make_async_copy(a, b, sm).wait()

        lin_issue(0, 0)
        lin_issue(1, 1)
        rsl = pl.ds(s * ROWS_PER_SUB, ROWS_PER_SUB)
        pltpu.sync_copy(x1_hbm.at[rsl], x1s.at[rsl])
        pltpu.sync_copy(zeros_hbm, acc.at[rsl])
        plsc.subcore_barrier()

        def process(kk, k, st):
            lin_wait(k, st)
            pltpu.async_copy(x1s.at[idx_s[st]], rows[st], semg[st])

            st2 = (st + 2) % 3

            @pl.when(kk + (1 if st > 0 else 0) > 0)
            def _():
                pltpu.make_async_copy(rows[st2], acc.at[idx_d[st2]], sems[st2]).wait()

            @pl.when(k + 2 < jw)
            def _():
                lin_issue(k + 2, st2)

            pltpu.make_async_copy(x1s.at[idx_s[st]], rows[st], semg[st]).wait()

            @plsc.parallel_loop(0, CH)
            def _(rp):
                for q in range(4):
                    sl = pl.ds(q * 16, 16)
                    rows[st][rp, sl] = rows[st][rp, sl] * wf[st][rp, sl]

            pltpu.async_copy(rows[st], acc.at[idx_d[st]], sems[st], add=True)

        def triple(kk, carry):
            for st in range(3):
                process(kk, 3 * kk + st, st)
            return carry

        lax.fori_loop(0, jw // 3, triple, 0)
        for t in range(jw % 3):
            k = jw - (jw % 3) + t
            process(jnp.int32(jw // 3), jnp.int32(k), k % 3)
        lst = (jw - 1) % 3
        pltpu.make_async_copy(rows[lst], acc.at[idx_d[lst]], sems[lst]).wait()
        plsc.subcore_barrier()
        pltpu.sync_copy(acc.at[rsl], out_hbm.at[c, rsl])

    return _sc_body


def kernel(x, r, edge_index, Wa, ba, Wd1, bd1, Wd2, bd2):
    f32 = jnp.float32
    ba2 = ba.reshape(1, D)
    bd1_2 = bd1.reshape(1, D)
    bd2_2 = bd2.reshape(1, D)

    x1 = pl.pallas_call(
        _x1_body,
        out_shape=jax.ShapeDtypeStruct((N_NODES, D), f32),
    )(x, Wa, ba2)

    npad = NEP - N_EDGES
    r3 = jnp.concatenate([r, jnp.full((npad,), 10.0, f32)]).reshape(NEP // TE, 1, TE)

    def wfilt_call(nb, i0):
        return pl.pallas_call(
            _wfilt_body,
            grid=(nb,),
            in_specs=[
                pl.BlockSpec((1, 1, TE), lambda i: (i + i0, 0, 0)),
                pl.BlockSpec((1, 1, TE), lambda i: (i + i0 + NBH, 0, 0)),
                pl.BlockSpec((NCEN, D), lambda i: (0, 0)),
                pl.BlockSpec((1, D), lambda i: (0, 0)),
                pl.BlockSpec((D, D), lambda i: (0, 0)),
                pl.BlockSpec((1, D), lambda i: (0, 0)),
            ],
            out_specs=pl.BlockSpec((TE, 128), lambda i: (i, 0)),
            out_shape=jax.ShapeDtypeStruct((nb * TE, 128), f32),
        )(r3, r3, Wd1[:NCEN], bd1_2, Wd2, bd2_2)

    wf_a = wfilt_call(NBA, 0)
    wf_b = wfilt_call(NBB, NBA)

    ipad = (jnp.arange(npad, dtype=edge_index.dtype) * 31) % N_NODES
    src = jnp.concatenate([edge_index[0], ipad])
    dst = jnp.concatenate([edge_index[1], ipad])
    zeros = jnp.zeros((ROWS_PER_SUB, D), f32)

    mesh = plsc.VectorSubcoreMesh(
        core_axis_name="c", subcore_axis_name="s",
        num_cores=NC, num_subcores=NS,
    )
    scr = (
        [pltpu.VMEM((CH,), jnp.int32)] * 6
        + [pltpu.VMEM((CH, D), f32)] * 6
        + [pltpu.VMEM_SHARED((N_NODES, D), f32)] * 2
        + [pltpu.SemaphoreType.DMA] * 9
    )

    def sc_call(body, wf):
        return pl.kernel(
            body,
            out_type=jax.ShapeDtypeStruct((NC, N_NODES, D), f32),
            mesh=mesh,
            scratch_types=list(scr),
            compiler_params=pltpu.CompilerParams(use_tc_tiling_on_sc=False),
        )(x1, src, dst, wf, zeros)

    p_a = sc_call(_make_sc_body(0, JWA), wf_a)
    p_b = sc_call(_make_sc_body(G0B, JWB), wf_b)

    out = pl.pallas_call(
        _out_body,
        out_shape=jax.ShapeDtypeStruct((N_NODES, D), f32),
    )(p_a, p_b, Wa, ba2)
    return out


# speedup vs baseline: 5.8753x; 1.1011x over previous
import jax
import jax.numpy as jnp
from jax import lax
from jax.experimental import pallas as pl
from jax.experimental.pallas import tpu as pltpu
from jax.experimental.pallas import tpu_sc as plsc

N_NODES = 10000
N_EDGES = 320000
D = 64
CUTOFF = 5.0

NEP = 335872
NEPH = NEP // 2
TE = 4096
NBH = NEPH // TE
CH = 128
WROWS = CH // 2
NCEN = 128
NC, NS = 2, 16
NW = NC * NS
JW = NEPH // WROWS // NW
NBA, NBB = 21, 20
JWA, JWB = 42, 40
G0B = NBA * TE // CH
ROWS_PER_SUB = N_NODES // NS


def _x1_body(x_ref, wa_ref, ba_ref, o_ref):
    o_ref[...] = (
        jnp.dot(x_ref[...], wa_ref[...], preferred_element_type=jnp.float32)
        + ba_ref[...]
    )


def _filter_half(r_row, wd1, bd1, wd2, bd2):
    cut_row = 0.5 * (jnp.cos(r_row * (jnp.pi / CUTOFF)) + 1.0)
    cut_row = jnp.where(r_row < CUTOFF, cut_row, 0.0)
    rr = jnp.transpose(r_row)
    centers = lax.broadcasted_iota(jnp.int32, (TE, NCEN), 1).astype(jnp.float32) * 0.1
    diff = rr - centers
    rbf = jnp.exp(-10.0 * diff * diff)
    h = jnp.tanh(jnp.dot(rbf, wd1, preferred_element_type=jnp.float32) + bd1)
    h = jnp.tanh(jnp.dot(h, wd2, preferred_element_type=jnp.float32) + bd2)
    return h, cut_row


def _wfilt_body(ra_ref, rb_ref, wd1_ref, bd1_ref, wd2_ref, bd2_ref,
                o_ref, ca_ref, cb_ref):
    wd1, bd1 = wd1_ref[...], bd1_ref[...]
    wd2, bd2 = wd2_ref[...], bd2_ref[...]
    ha, cuta = _filter_half(ra_ref[0], wd1, bd1, wd2, bd2)
    hb, cutb = _filter_half(rb_ref[0], wd1, bd1, wd2, bd2)
    o_ref[...] = jnp.concatenate([ha, hb], axis=1)
    ca_ref[...] = cuta.reshape(1, 1, TE)
    cb_ref[...] = cutb.reshape(1, 1, TE)


def _out_body(p_ref, q_ref, wa_ref, ba_ref, o_ref):
    prop = p_ref[0] + p_ref[1] + q_ref[0] + q_ref[1]
    x2 = jnp.tanh(
        jnp.dot(prop, wa_ref[...], preferred_element_type=jnp.float32)
        + ba_ref[...]
    )
    o_ref[...] = (
        jnp.dot(x2, wa_ref[...], preferred_element_type=jnp.float32)
        + ba_ref[...]
    )


def _make_sc_body(g0, jw):
    def _sc_body(x1_hbm, src_hbm, dst_hbm, wf_hbm, cuta_hbm, cutb_hbm,
                 zeros_hbm, out_hbm,
                 idx_s0, idx_s1, idx_s2, idx_d0, idx_d1, idx_d2,
                 rows0, rows1, rows2, wf0, wf1, wf2,
                 cut0, cut1, cut2,
                 x1s, acc,
                 seml0, seml1, seml2, semg0, semg1, semg2,
                 sems0, sems1, sems2):
        c = lax.axis_index("c")
        s = lax.axis_index("s")
        wid = s * NC + c
        half = wid // 16
        wsub = wid % 16

        idx_s = (idx_s0, idx_s1, idx_s2)
        idx_d = (idx_d0, idx_d1, idx_d2)
        rows = (rows0, rows1, rows2)
        wf = (wf0, wf1, wf2)
        cut = (cut0, cut1, cut2)
        seml = (seml0, seml1, seml2)
        semg = (semg0, semg1, semg2)
        sems = (sems0, sems1, sems2)

        def lin_copies(k, st):
            cbase = pl.multiple_of((wsub * jw + k) * CH, CH)
            eb = pl.multiple_of(half * NEPH + g0 * CH + cbase, CH)
            return (
                (src_hbm.at[pl.ds(eb, CH)], idx_s[st], seml[st]),
                (dst_hbm.at[pl.ds(eb, CH)], idx_d[st], seml[st]),
                (wf_hbm.at[pl.ds(cbase, CH), pl.ds(half * D, D)], wf[st], seml[st]),
            )

        def lin_issue(k, st):
            for a, b, sm in lin_copies(k, st):
                pltpu.async_copy(a, b, sm)
            cbase = pl.multiple_of((wsub * jw + k) * CH, CH)

            @pl.when(half == 0)
            def _():
                pltpu.async_copy(cuta_hbm.at[pl.ds(cbase, CH)], cut[st], seml[st])

            @pl.when(half == 1)
            def _():
                pltpu.async_copy(cutb_hbm.at[pl.ds(cbase, CH)], cut[st], seml[st])

        def lin_wait(k, st):
            for a, b, sm in lin_copies(k, st):
                pltpu.make_async_copy(a, b, sm).wait()
            cbase = pl.multiple_of((wsub * jw + k) * CH, CH)
            pltpu.make_async_copy(
                cuta_hbm.at[pl.ds(cbase, CH)], cut[st], seml[st]).wait()

        lin_issue(0, 0)
        lin_issue(1, 1)
        rsl = pl.ds(s * ROWS_PER_SUB, ROWS_PER_SUB)
        pltpu.sync_copy(x1_hbm.at[rsl], x1s.at[rsl])
        pltpu.sync_copy(zeros_hbm, acc.at[rsl])
        plsc.subcore_barrier()

        def process(kk, k, st):
            lin_wait(k, st)
            pltpu.async_copy(x1s.at[idx_s[st]], rows[st], semg[st])

            st2 = (st + 2) % 3

            @pl.when(kk + (1 if st > 0 else 0) > 0)
            def _():
                pltpu.make_async_copy(rows[st2], acc.at[idx_d[st2]], sems[st2]).wait()

            @pl.when(k + 2 < jw)
            def _():
                lin_issue(k + 2, st2)

            pltpu.make_async_copy(x1s.at[idx_s[st]], rows[st], semg[st]).wait()

            def mul_group(gp, mc):
                cv = cut[st][pl.ds(gp * 16, 16)]
                for i in range(16):
                    rp = gp * 16 + i
                    ci = cv[i]
                    for q in range(4):
                        sl = pl.ds(q * 16, 16)
                        rows[st][rp, sl] = rows[st][rp, sl] * (wf[st][rp, sl] * ci)
                return mc

            lax.fori_loop(0, CH // 16, mul_group, 0)

            pltpu.async_copy(rows[st], acc.at[idx_d[st]], sems[st], add=True)

        def triple(kk, carry):
            for st in range(3):
                process(kk, 3 * kk + st, st)
            return carry

        lax.fori_loop(0, jw // 3, triple, 0)
        for t in range(jw % 3):
            k = jw - (jw % 3) + t
            process(jnp.int32(jw // 3), jnp.int32(k), k % 3)
        lst = (jw - 1) % 3
        pltpu.make_async_copy(rows[lst], acc.at[idx_d[lst]], sems[lst]).wait()
        plsc.subcore_barrier()
        pltpu.sync_copy(acc.at[rsl], out_hbm.at[c, rsl])

    return _sc_body


def kernel(x, r, edge_index, Wa, ba, Wd1, bd1, Wd2, bd2):
    f32 = jnp.float32
    ba2 = ba.reshape(1, D)
    bd1_2 = bd1.reshape(1, D)
    bd2_2 = bd2.reshape(1, D)

    x1 = pl.pallas_call(
        _x1_body,
        out_shape=jax.ShapeDtypeStruct((N_NODES, D), f32),
    )(x, Wa, ba2)

    npad = NEP - N_EDGES
    r3 = jnp.concatenate([r, jnp.full((npad,), 10.0, f32)]).reshape(NEP // TE, 1, TE)

    def wfilt_call(nb, i0):
        wf, ca, cb = pl.pallas_call(
            _wfilt_body,
            grid=(nb,),
            in_specs=[
                pl.BlockSpec((1, 1, TE), lambda i: (i + i0, 0, 0)),
                pl.BlockSpec((1, 1, TE), lambda i: (i + i0 + NBH, 0, 0)),
                pl.BlockSpec((NCEN, D), lambda i: (0, 0)),
                pl.BlockSpec((1, D), lambda i: (0, 0)),
                pl.BlockSpec((D, D), lambda i: (0, 0)),
                pl.BlockSpec((1, D), lambda i: (0, 0)),
            ],
            out_specs=[
                pl.BlockSpec((TE, 128), lambda i: (i, 0)),
                pl.BlockSpec((1, 1, TE), lambda i: (i, 0, 0)),
                pl.BlockSpec((1, 1, TE), lambda i: (i, 0, 0)),
            ],
            out_shape=[
                jax.ShapeDtypeStruct((nb * TE, 128), f32),
                jax.ShapeDtypeStruct((nb, 1, TE), f32),
                jax.ShapeDtypeStruct((nb, 1, TE), f32),
            ],
        )(r3, r3, Wd1[:NCEN], bd1_2, Wd2, bd2_2)
        return wf, ca.reshape(nb * TE), cb.reshape(nb * TE)

    wf_a, cuta_a, cutb_a = wfilt_call(NBA, 0)
    wf_b, cuta_b, cutb_b = wfilt_call(NBB, NBA)

    ipad = (jnp.arange(npad, dtype=edge_index.dtype) * 31) % N_NODES
    src = jnp.concatenate([edge_index[0], ipad])
    dst = jnp.concatenate([edge_index[1], ipad])
    zeros = jnp.zeros((ROWS_PER_SUB, D), f32)

    mesh = plsc.VectorSubcoreMesh(
        core_axis_name="c", subcore_axis_name="s",
        num_cores=NC, num_subcores=NS,
    )
    scr = (
        [pltpu.VMEM((CH,), jnp.int32)] * 6
        + [pltpu.VMEM((CH, D), f32)] * 6
        + [pltpu.VMEM((CH,), f32)] * 3
        + [pltpu.VMEM_SHARED((N_NODES, D), f32)] * 2
        + [pltpu.SemaphoreType.DMA] * 9
    )

    def sc_call(body, wf, cuta, cutb):
        return pl.kernel(
            body,
            out_type=jax.ShapeDtypeStruct((NC, N_NODES, D), f32),
            mesh=mesh,
            scratch_types=list(scr),
            compiler_params=pltpu.CompilerParams(use_tc_tiling_on_sc=False),
        )(x1, src, dst, wf, cuta, cutb, zeros)

    p_a = sc_call(_make_sc_body(0, JWA), wf_a, cuta_a, cutb_a)
    p_b = sc_call(_make_sc_body(G0B, JWB), wf_b, cuta_b, cutb_b)

    out = pl.pallas_call(
        _out_body,
        out_shape=jax.ShapeDtypeStruct((N_NODES, D), f32),
    )(p_a, p_b, Wa, ba2)
    return out


# speedup vs baseline: 6.1571x; 1.0480x over previous
import jax
import jax.numpy as jnp
from jax import lax
from jax.experimental import pallas as pl
from jax.experimental.pallas import tpu as pltpu
from jax.experimental.pallas import tpu_sc as plsc

N_NODES = 10000
N_EDGES = 320000
D = 64
CUTOFF = 5.0

NEP = 335872
NEPH = NEP // 2
TE = 4096
NBH = NEPH // TE
CH = 128
WROWS = CH // 2
NCEN = 128
NC, NS = 2, 16
NW = NC * NS
JW = NEPH // WROWS // NW
NBA, NBB = 21, 20
JWA, JWB = 42, 40
G0B = NBA * TE // CH
ROWS_PER_SUB = N_NODES // NS


def _x1_body(x_ref, wa_ref, ba_ref, o_ref):
    o_ref[...] = (
        jnp.dot(x_ref[...], wa_ref[...], preferred_element_type=jnp.float32)
        + ba_ref[...]
    )


def _filter_half(r_row, wd1, bd1, wd2, bd2):
    cut_row = 0.5 * (jnp.cos(r_row * (jnp.pi / CUTOFF)) + 1.0)
    cut_row = jnp.where(r_row < CUTOFF, cut_row, 0.0)
    rr = jnp.transpose(r_row)
    centers = lax.broadcasted_iota(jnp.int32, (TE, NCEN), 1).astype(jnp.float32) * 0.1
    diff = rr - centers
    rbf = jnp.exp(-10.0 * diff * diff)
    h = jnp.tanh(jnp.dot(rbf, wd1, preferred_element_type=jnp.float32) + bd1)
    h = jnp.tanh(jnp.dot(h, wd2, preferred_element_type=jnp.float32) + bd2)
    return h, cut_row


def _wfilt_body(ra_ref, rb_ref, wd1_ref, bd1_ref, wd2_ref, bd2_ref,
                o_ref, ca_ref, cb_ref):
    wd1, bd1 = wd1_ref[...], bd1_ref[...]
    wd2, bd2 = wd2_ref[...], bd2_ref[...]
    ha, cuta = _filter_half(ra_ref[0], wd1, bd1, wd2, bd2)
    hb, cutb = _filter_half(rb_ref[0], wd1, bd1, wd2, bd2)
    o_ref[...] = jnp.concatenate([ha, hb], axis=1)
    ca_ref[...] = cuta.reshape(1, 1, TE)
    cb_ref[...] = cutb.reshape(1, 1, TE)


def _out_body(p_ref, q_ref, wa_ref, ba_ref, o_ref):
    prop = p_ref[0] + p_ref[1] + q_ref[0] + q_ref[1]
    x2 = jnp.tanh(
        jnp.dot(prop, wa_ref[...], preferred_element_type=jnp.float32)
        + ba_ref[...]
    )
    o_ref[...] = (
        jnp.dot(x2, wa_ref[...], preferred_element_type=jnp.float32)
        + ba_ref[...]
    )


def _make_sc_body(g0, jw):
    def _sc_body(x1_hbm, ei_hbm, wf_hbm, cuta_hbm, cutb_hbm,
                 zeros_hbm, out_hbm,
                 idx_s0, idx_s1, idx_s2, idx_d0, idx_d1, idx_d2,
                 rows0, rows1, rows2, wf0, wf1, wf2,
                 cut0, cut1, cut2,
                 x1s, acc,
                 seml0, seml1, seml2, semg0, semg1, semg2,
                 sems0, sems1, sems2):
        c = lax.axis_index("c")
        s = lax.axis_index("s")
        wid = s * NC + c
        half = wid // 16
        wsub = wid % 16

        idx_s = (idx_s0, idx_s1, idx_s2)
        idx_d = (idx_d0, idx_d1, idx_d2)
        rows = (rows0, rows1, rows2)
        wf = (wf0, wf1, wf2)
        cut = (cut0, cut1, cut2)
        seml = (seml0, seml1, seml2)
        semg = (semg0, semg1, semg2)
        sems = (sems0, sems1, sems2)

        base0 = half * NEPH + g0 * CH + wsub * jw * CH
        nproc = jnp.minimum(jnp.maximum((N_EDGES - base0) // CH, 0), jw)

        def lin_copies(k, st):
            cbase = pl.multiple_of((wsub * jw + k) * CH, CH)
            eb = pl.multiple_of(half * NEPH + g0 * CH + cbase, CH)
            return (
                (ei_hbm.at[0, pl.ds(eb, CH)], idx_s[st], seml[st]),
                (ei_hbm.at[1, pl.ds(eb, CH)], idx_d[st], seml[st]),
                (wf_hbm.at[pl.ds(cbase, CH), pl.ds(half * D, D)], wf[st], seml[st]),
            )

        def lin_issue(k, st):
            for a, b, sm in lin_copies(k, st):
                pltpu.async_copy(a, b, sm)
            cbase = pl.multiple_of((wsub * jw + k) * CH, CH)

            @pl.when(half == 0)
            def _():
                pltpu.async_copy(cuta_hbm.at[pl.ds(cbase, CH)], cut[st], seml[st])

            @pl.when(half == 1)
            def _():
                pltpu.async_copy(cutb_hbm.at[pl.ds(cbase, CH)], cut[st], seml[st])

        def lin_wait(k, st):
            for a, b, sm in lin_copies(k, st):
                pltpu.make_async_copy(a, b, sm).wait()
            cbase = pl.multiple_of((wsub * jw + k) * CH, CH)
            pltpu.make_async_copy(
                cuta_hbm.at[pl.ds(cbase, CH)], cut[st], seml[st]).wait()

        @pl.when(nproc > 0)
        def _():
            lin_issue(0, 0)

        @pl.when(nproc > 1)
        def _():
            lin_issue(1, 1)

        rsl = pl.ds(s * ROWS_PER_SUB, ROWS_PER_SUB)
        pltpu.sync_copy(x1_hbm.at[rsl], x1s.at[rsl])
        pltpu.sync_copy(zeros_hbm, acc.at[rsl])
        plsc.subcore_barrier()

        def process(kk, k, st):
            @pl.when(k < nproc)
            def _():
                lin_wait(k, st)
                pltpu.async_copy(x1s.at[idx_s[st]], rows[st], semg[st])

                st2 = (st + 2) % 3

                @pl.when(kk + (1 if st > 0 else 0) > 0)
                def _():
                    pltpu.make_async_copy(rows[st2], acc.at[idx_d[st2]], sems[st2]).wait()

                @pl.when(k + 2 < jnp.minimum(nproc, jw))
                def _():
                    lin_issue(k + 2, st2)

                pltpu.make_async_copy(x1s.at[idx_s[st]], rows[st], semg[st]).wait()

                def mul_group(gp, mc):
                    cv = cut[st][pl.ds(gp * 16, 16)]
                    for i in range(16):
                        rp = gp * 16 + i
                        ci = cv[i]
                        for q in range(4):
                            sl = pl.ds(q * 16, 16)
                            rows[st][rp, sl] = rows[st][rp, sl] * (wf[st][rp, sl] * ci)
                    return mc

                lax.fori_loop(0, CH // 16, mul_group, 0)

                pltpu.async_copy(rows[st], acc.at[idx_d[st]], sems[st], add=True)

        def triple(kk, carry):
            for st in range(3):
                process(kk, 3 * kk + st, st)
            return carry

        lax.fori_loop(0, jw // 3, triple, 0)
        for t in range(jw % 3):
            k = jw - (jw % 3) + t
            process(jnp.int32(jw // 3), jnp.int32(k), k % 3)
        for st in range(3):
            @pl.when((nproc > 0) & ((nproc - 1) % 3 == st))
            def _():
                pltpu.make_async_copy(rows[st], acc.at[idx_d[st]], sems[st]).wait()
        plsc.subcore_barrier()
        pltpu.sync_copy(acc.at[rsl], out_hbm.at[c, rsl])

    return _sc_body


def kernel(x, r, edge_index, Wa, ba, Wd1, bd1, Wd2, bd2):
    f32 = jnp.float32
    ba2 = ba.reshape(1, D)
    bd1_2 = bd1.reshape(1, D)
    bd2_2 = bd2.reshape(1, D)

    x1 = pl.pallas_call(
        _x1_body,
        out_shape=jax.ShapeDtypeStruct((N_NODES, D), f32),
    )(x, Wa, ba2)

    npad = NEP - N_EDGES
    r3 = jnp.concatenate([r, jnp.full((npad,), 10.0, f32)]).reshape(NEP // TE, 1, TE)

    def wfilt_call(nb, i0):
        wf, ca, cb = pl.pallas_call(
            _wfilt_body,
            grid=(nb,),
            in_specs=[
                pl.BlockSpec((1, 1, TE), lambda i: (i + i0, 0, 0)),
                pl.BlockSpec((1, 1, TE), lambda i: (i + i0 + NBH, 0, 0)),
                pl.BlockSpec((NCEN, D), lambda i: (0, 0)),
                pl.BlockSpec((1, D), lambda i: (0, 0)),
                pl.BlockSpec((D, D), lambda i: (0, 0)),
                pl.BlockSpec((1, D), lambda i: (0, 0)),
            ],
            out_specs=[
                pl.BlockSpec((TE, 128), lambda i: (i, 0)),
                pl.BlockSpec((1, 1, TE), lambda i: (i, 0, 0)),
                pl.BlockSpec((1, 1, TE), lambda i: (i, 0, 0)),
            ],
            out_shape=[
                jax.ShapeDtypeStruct((nb * TE, 128), f32),
                jax.ShapeDtypeStruct((nb, 1, TE), f32),
                jax.ShapeDtypeStruct((nb, 1, TE), f32),
            ],
        )(r3, r3, Wd1[:NCEN], bd1_2, Wd2, bd2_2)
        return wf, ca.reshape(nb * TE), cb.reshape(nb * TE)

    wf_a, cuta_a, cutb_a = wfilt_call(NBA, 0)
    wf_b, cuta_b, cutb_b = wfilt_call(NBB, NBA)

    zeros = jnp.zeros((ROWS_PER_SUB, D), f32)

    mesh = plsc.VectorSubcoreMesh(
        core_axis_name="c", subcore_axis_name="s",
        num_cores=NC, num_subcores=NS,
    )
    scr = (
        [pltpu.VMEM((CH,), jnp.int32)] * 6
        + [pltpu.VMEM((CH, D), f32)] * 6
        + [pltpu.VMEM((CH,), f32)] * 3
        + [pltpu.VMEM_SHARED((N_NODES, D), f32)] * 2
        + [pltpu.SemaphoreType.DMA] * 9
    )

    def sc_call(body, wf, cuta, cutb):
        return pl.kernel(
            body,
            out_type=jax.ShapeDtypeStruct((NC, N_NODES, D), f32),
            mesh=mesh,
            scratch_types=list(scr),
            compiler_params=pltpu.CompilerParams(use_tc_tiling_on_sc=False),
        )(x1, edge_index, wf, cuta, cutb, zeros)

    p_a = sc_call(_make_sc_body(0, JWA), wf_a, cuta_a, cutb_a)
    p_b = sc_call(_make_sc_body(G0B, JWB), wf_b, cuta_b, cutb_b)

    out = pl.pallas_call(
        _out_body,
        out_shape=jax.ShapeDtypeStruct((N_NODES, D), f32),
    )(p_a, p_b, Wa, ba2)
    return out
